# baseline (speedup 1.0000x reference)
import functools

import jax
import jax.numpy as jnp
from jax import lax
from jax.experimental import pallas as pl
from jax.experimental.pallas import tpu as pltpu
from jax.experimental.pallas import tpu_sc as plsc

_NUM_CORES = 2
_NUM_SUBCORES = 16
_NUM_WORKERS = _NUM_CORES * _NUM_SUBCORES
_GATHER_CHUNK = 16


def _gather_body(n_tokens, hidden, table_hbm, idx_hbm, out_hbm,
                 idx_v, rows_a, rows_b, sem_a, sem_b, sem_out):
    b_per_w = n_tokens // _NUM_WORKERS
    wid = lax.axis_index("s") * _NUM_CORES + lax.axis_index("c")
    base = wid * b_per_w
    pltpu.sync_copy(idx_hbm.at[pl.ds(base, b_per_w)], idx_v)
    n_chunks = b_per_w // _GATHER_CHUNK
    bufs = [(rows_a, sem_a), (rows_b, sem_b)]

    def gather(c):
        buf, sem = bufs[c % 2]
        return pltpu.async_copy(
            table_hbm.at[idx_v.at[pl.ds(c * _GATHER_CHUNK, _GATHER_CHUNK)]],
            buf, sem)

    in_flight = gather(0)
    out_handles = []
    for c in range(n_chunks):
        buf, _ = bufs[c % 2]
        in_flight.wait()
        if c + 1 < n_chunks:
            in_flight = gather(c + 1)
        out_handles.append(pltpu.async_copy(
            buf, out_hbm.at[pl.ds(base + c * _GATHER_CHUNK, _GATHER_CHUNK)],
            sem_out))
    for h in out_handles:
        h.wait()


def _sc_gather(table, ids):
    n_tokens, = ids.shape
    vocab, hidden = table.shape
    b_per_w = n_tokens // _NUM_WORKERS
    mesh = plsc.VectorSubcoreMesh(core_axis_name="c", subcore_axis_name="s")
    kern = pl.kernel(
        functools.partial(_gather_body, n_tokens, hidden),
        out_type=jax.ShapeDtypeStruct((n_tokens, hidden), table.dtype),
        mesh=mesh,
        scratch_types=[
            pltpu.VMEM((b_per_w,), jnp.int32),
            pltpu.VMEM((_GATHER_CHUNK, hidden), table.dtype),
            pltpu.VMEM((_GATHER_CHUNK, hidden), table.dtype),
            pltpu.SemaphoreType.DMA,
            pltpu.SemaphoreType.DMA,
            pltpu.SemaphoreType.DMA,
        ],
    )
    return kern(table, ids)


_M_TILE = 2048
_X_SLICE = 512


def _mm_body(x_hbm, w_ref, b_ref, o_ref, xbf, xf32_a, xf32_b, sem_a, sem_b):
    m, k = xbf.shape
    j = pl.program_id(0)
    wb = w_ref[...].astype(jnp.bfloat16)
    bias = b_ref[...]

    def dot_rows(sl):
        o_ref[sl, :] = lax.dot_general(
            xbf[sl, :], wb,
            dimension_numbers=(((1,), (1,)), ((), ())),
            preferred_element_type=jnp.float32) + bias

    @pl.when(j == 0)
    def _prologue():
        bufs = [(xf32_a, sem_a), (xf32_b, sem_b)]
        n_sl = m // _X_SLICE

        def start(s):
            buf, sem = bufs[s % 2]
            copy = pltpu.make_async_copy(
                x_hbm.at[pl.ds(s * _X_SLICE, _X_SLICE), :], buf, sem)
            copy.start()
            return copy

        in_flight = start(0)
        for s in range(n_sl):
            buf, _ = bufs[s % 2]
            in_flight.wait()
            if s + 1 < n_sl:
                in_flight = start(s + 1)
            xbf[pl.ds(s * _X_SLICE, _X_SLICE), :] = buf[...].astype(
                jnp.bfloat16)

    for mi in range(m // _M_TILE):
        dot_rows(pl.ds(mi * _M_TILE, _M_TILE))


def _tc_matmul(x, w, b2d, n_blk):
    m, k = x.shape
    v, _ = w.shape
    grid = (v // n_blk,)
    return pl.pallas_call(
        _mm_body,
        grid=grid,
        in_specs=[
            pl.BlockSpec(memory_space=pl.ANY),
            pl.BlockSpec((n_blk, k), lambda j: (j, 0)),
            pl.BlockSpec((1, n_blk), lambda j: (0, j)),
        ],
        out_specs=pl.BlockSpec((m, n_blk), lambda j: (0, j)),
        out_shape=jax.ShapeDtypeStruct((m, v), jnp.float32),
        scratch_shapes=[
            pltpu.VMEM((m, k), jnp.bfloat16),
            pltpu.VMEM((_X_SLICE, k), jnp.float32),
            pltpu.VMEM((_X_SLICE, k), jnp.float32),
            pltpu.SemaphoreType.DMA,
            pltpu.SemaphoreType.DMA,
        ],
        compiler_params=pltpu.CompilerParams(
            dimension_semantics=("arbitrary",),
        ),
    )(x, w, b2d)


def kernel(input_ids, embedding, W, b):
    batch, seq = input_ids.shape
    vocab, hidden = W.shape
    ids = input_ids.reshape(-1).astype(jnp.int32)
    x = _sc_gather(embedding, ids)
    out = _tc_matmul(x, W, b.reshape(1, vocab), n_blk=256)
    return out.reshape(batch, seq, vocab)


# speedup vs baseline: 1.0203x; 1.0203x over previous
import functools

import jax
import jax.numpy as jnp
from jax import lax
from jax.experimental import pallas as pl
from jax.experimental.pallas import tpu as pltpu
from jax.experimental.pallas import tpu_sc as plsc

_NUM_CORES = 2
_NUM_SUBCORES = 16
_NUM_WORKERS = _NUM_CORES * _NUM_SUBCORES
_GATHER_CHUNK = 16


def _gather_body(n_tokens, hidden, table_hbm, idx_hbm, out_hbm,
                 idx_v, rows_a, rows_b, sem_a, sem_b, sem_out):
    b_per_w = n_tokens // _NUM_WORKERS
    wid = lax.axis_index("s") * _NUM_CORES + lax.axis_index("c")
    base = wid * b_per_w
    pltpu.sync_copy(idx_hbm.at[pl.ds(base, b_per_w)], idx_v)
    n_chunks = b_per_w // _GATHER_CHUNK
    bufs = [(rows_a, sem_a), (rows_b, sem_b)]

    def gather(c):
        buf, sem = bufs[c % 2]
        return pltpu.async_copy(
            table_hbm.at[idx_v.at[pl.ds(c * _GATHER_CHUNK, _GATHER_CHUNK)]],
            buf, sem)

    in_flight = gather(0)
    out_handles = []
    for c in range(n_chunks):
        buf, _ = bufs[c % 2]
        in_flight.wait()
        if c + 1 < n_chunks:
            in_flight = gather(c + 1)
        out_handles.append(pltpu.async_copy(
            buf, out_hbm.at[pl.ds(base + c * _GATHER_CHUNK, _GATHER_CHUNK)],
            sem_out))
    for h in out_handles:
        h.wait()


def _sc_gather(table, ids):
    n_tokens, = ids.shape
    vocab, hidden = table.shape
    b_per_w = n_tokens // _NUM_WORKERS
    mesh = plsc.VectorSubcoreMesh(core_axis_name="c", subcore_axis_name="s")
    kern = pl.kernel(
        functools.partial(_gather_body, n_tokens, hidden),
        out_type=jax.ShapeDtypeStruct((n_tokens, hidden), table.dtype),
        mesh=mesh,
        scratch_types=[
            pltpu.VMEM((b_per_w,), jnp.int32),
            pltpu.VMEM((_GATHER_CHUNK, hidden), table.dtype),
            pltpu.VMEM((_GATHER_CHUNK, hidden), table.dtype),
            pltpu.SemaphoreType.DMA,
            pltpu.SemaphoreType.DMA,
            pltpu.SemaphoreType.DMA,
        ],
    )
    return kern(table, ids)


_M_TILE = 2048
_X_SLICE = 512


def _mm_body(x_hbm, w_ref, b_ref, o_ref, xbf, xf32_a, xf32_b, sem_a, sem_b):
    m, k = xbf.shape
    j = pl.program_id(0)

    @pl.when(j == 0)
    def _prologue():
        bufs = [(xf32_a, sem_a), (xf32_b, sem_b)]
        n_sl = m // _X_SLICE

        def start(s):
            buf, sem = bufs[s % 2]
            copy = pltpu.make_async_copy(
                x_hbm.at[pl.ds(s * _X_SLICE, _X_SLICE), :], buf, sem)
            copy.start()
            return copy

        in_flight = start(0)
        for s in range(n_sl):
            buf, _ = bufs[s % 2]
            in_flight.wait()
            if s + 1 < n_sl:
                in_flight = start(s + 1)
            xbf[pl.ds(s * _X_SLICE, _X_SLICE), :] = buf[...].astype(
                jnp.bfloat16)

    wb = w_ref[...].astype(jnp.bfloat16)
    bias = b_ref[...]
    for mi in range(m // _M_TILE):
        sl = pl.ds(mi * _M_TILE, _M_TILE)
        o_ref[sl, :] = lax.dot_general(
            xbf[sl, :], wb,
            dimension_numbers=(((1,), (1,)), ((), ())),
            preferred_element_type=jnp.float32) + bias


def _tc_matmul(x, w, b2d, n_blk):
    m, k = x.shape
    v, _ = w.shape
    grid = (v // n_blk,)
    return pl.pallas_call(
        _mm_body,
        grid=grid,
        in_specs=[
            pl.BlockSpec(memory_space=pl.ANY),
            pl.BlockSpec((n_blk, k), lambda j: (j, 0)),
            pl.BlockSpec((1, n_blk), lambda j: (0, j)),
        ],
        out_specs=pl.BlockSpec((m, n_blk), lambda j: (0, j)),
        out_shape=jax.ShapeDtypeStruct((m, v), jnp.float32),
        scratch_shapes=[
            pltpu.VMEM((m, k), jnp.bfloat16),
            pltpu.VMEM((_X_SLICE, k), jnp.float32),
            pltpu.VMEM((_X_SLICE, k), jnp.float32),
            pltpu.SemaphoreType.DMA,
            pltpu.SemaphoreType.DMA,
        ],
        compiler_params=pltpu.CompilerParams(
            dimension_semantics=("arbitrary",),
        ),
    )(x, w, b2d)


def kernel(input_ids, embedding, W, b):
    batch, seq = input_ids.shape
    vocab, hidden = W.shape
    ids = input_ids.reshape(-1).astype(jnp.int32)
    x = _sc_gather(embedding, ids)
    out = _tc_matmul(x, W, b.reshape(1, vocab), n_blk=256)
    return out.reshape(batch, seq, vocab)


# speedup vs baseline: 1.0219x; 1.0015x over previous
import functools

import jax
import jax.numpy as jnp
from jax import lax
from jax.experimental import pallas as pl
from jax.experimental.pallas import tpu as pltpu
from jax.experimental.pallas import tpu_sc as plsc

_NUM_CORES = 2
_NUM_SUBCORES = 16
_NUM_WORKERS = _NUM_CORES * _NUM_SUBCORES
_GATHER_CHUNK = 16


def _gather_body(n_tokens, hidden, table_hbm, idx_hbm, out_hbm,
                 idx_v, rows_a, rows_b, rows_c, sem_a, sem_b, sem_c, sem_out):
    b_per_w = n_tokens // _NUM_WORKERS
    wid = lax.axis_index("s") * _NUM_CORES + lax.axis_index("c")
    base = wid * b_per_w
    pltpu.sync_copy(idx_hbm.at[pl.ds(base, b_per_w)], idx_v)
    n_chunks = b_per_w // _GATHER_CHUNK
    bufs = [(rows_a, sem_a), (rows_b, sem_b), (rows_c, sem_c)]

    def gather(c):
        buf, sem = bufs[c % 3]
        return pltpu.async_copy(
            table_hbm.at[idx_v.at[pl.ds(c * _GATHER_CHUNK, _GATHER_CHUNK)]],
            buf, sem)

    in_flight = [gather(0), gather(1)]
    out_handles = []
    for c in range(n_chunks):
        buf, _ = bufs[c % 3]
        in_flight.pop(0).wait()
        if c + 2 < n_chunks:
            in_flight.append(gather(c + 2))
        out_handles.append(pltpu.async_copy(
            buf, out_hbm.at[pl.ds(base + c * _GATHER_CHUNK, _GATHER_CHUNK)],
            sem_out))
    for h in out_handles:
        h.wait()


def _sc_gather(table, ids):
    n_tokens, = ids.shape
    vocab, hidden = table.shape
    b_per_w = n_tokens // _NUM_WORKERS
    mesh = plsc.VectorSubcoreMesh(core_axis_name="c", subcore_axis_name="s")
    kern = pl.kernel(
        functools.partial(_gather_body, n_tokens, hidden),
        out_type=jax.ShapeDtypeStruct((n_tokens, hidden), table.dtype),
        mesh=mesh,
        scratch_types=[
            pltpu.VMEM((b_per_w,), jnp.int32),
            pltpu.VMEM((_GATHER_CHUNK, hidden), table.dtype),
            pltpu.VMEM((_GATHER_CHUNK, hidden), table.dtype),
            pltpu.VMEM((_GATHER_CHUNK, hidden), table.dtype),
            pltpu.SemaphoreType.DMA,
            pltpu.SemaphoreType.DMA,
            pltpu.SemaphoreType.DMA,
            pltpu.SemaphoreType.DMA,
        ],
    )
    return kern(table, ids)


_M_TILE = 2048
_X_SLICE = 512


def _mm_body(x_hbm, w_ref, b_ref, o_ref, xbf, xf32_a, xf32_b, sem_a, sem_b):
    m, k = xbf.shape
    j = pl.program_id(0)

    @pl.when(j == 0)
    def _prologue():
        bufs = [(xf32_a, sem_a), (xf32_b, sem_b)]
        n_sl = m // _X_SLICE

        def start(s):
            buf, sem = bufs[s % 2]
            copy = pltpu.make_async_copy(
                x_hbm.at[pl.ds(s * _X_SLICE, _X_SLICE), :], buf, sem)
            copy.start()
            return copy

        in_flight = start(0)
        for s in range(n_sl):
            buf, _ = bufs[s % 2]
            in_flight.wait()
            if s + 1 < n_sl:
                in_flight = start(s + 1)
            xbf[pl.ds(s * _X_SLICE, _X_SLICE), :] = buf[...].astype(
                jnp.bfloat16)

    wb = w_ref[...].astype(jnp.bfloat16)
    bias = b_ref[...]
    for mi in range(m // _M_TILE):
        sl = pl.ds(mi * _M_TILE, _M_TILE)
        o_ref[sl, :] = lax.dot_general(
            xbf[sl, :], wb,
            dimension_numbers=(((1,), (1,)), ((), ())),
            preferred_element_type=jnp.float32) + bias


def _tc_matmul(x, w, b2d, n_blk):
    m, k = x.shape
    v, _ = w.shape
    grid = (v // n_blk,)
    return pl.pallas_call(
        _mm_body,
        grid=grid,
        in_specs=[
            pl.BlockSpec(memory_space=pl.ANY),
            pl.BlockSpec((n_blk, k), lambda j: (j, 0)),
            pl.BlockSpec((1, n_blk), lambda j: (0, j)),
        ],
        out_specs=pl.BlockSpec((m, n_blk), lambda j: (0, j)),
        out_shape=jax.ShapeDtypeStruct((m, v), jnp.float32),
        scratch_shapes=[
            pltpu.VMEM((m, k), jnp.bfloat16),
            pltpu.VMEM((_X_SLICE, k), jnp.float32),
            pltpu.VMEM((_X_SLICE, k), jnp.float32),
            pltpu.SemaphoreType.DMA,
            pltpu.SemaphoreType.DMA,
        ],
        compiler_params=pltpu.CompilerParams(
            dimension_semantics=("arbitrary",),
        ),
    )(x, w, b2d)


def kernel(input_ids, embedding, W, b):
    batch, seq = input_ids.shape
    vocab, hidden = W.shape
    ids = input_ids.reshape(-1).astype(jnp.int32)
    x = _sc_gather(embedding, ids)
    out = _tc_matmul(x, W, b.reshape(1, vocab), n_blk=256)
    return out.reshape(batch, seq, vocab)
